# baseline (speedup 1.0000x reference)
import functools
import math

import jax
import jax.numpy as jnp
from jax import lax
from jax.experimental import pallas as pl
from jax.experimental.pallas import tpu as pltpu
from jax.experimental.pallas import tpu_sc as plsc

_DIM = 256
_H = 2
_DH = _DIM // _H
_HALF = 128
_NC = 2
_NS = 16


def _fill_zero_ones(stage, ones):
    zero16 = jnp.zeros((16,), jnp.float32)
    one16 = jnp.ones((16,), jnp.float32)

    def fill(r, _):
        for g in range(_HALF // 16):
            stage[r, pl.ds(g * 16, 16)] = zero16
            if ones is not None:
                ones[r, pl.ds(g * 16, 16)] = one16
        return 0
    lax.fori_loop(0, 128, fill, 0)


def _clamp_dst(didx, steps, rbase, nrows):
    def clampd(j, _):
        for g in range(128 // 16):
            v = didx[j, pl.ds(g * 16, 16)] - rbase
            ok = (v >= 0) & (v < nrows)
            didx[j, pl.ds(g * 16, 16)] = jnp.where(ok, v, nrows)
        return 0
    lax.fori_loop(0, steps, clampd, 0)


@functools.lru_cache(maxsize=None)
def _make_seg_fused(n_dst, n_edges, nconv):
    halves = 2 if n_dst * _HALF * 4 > 3 * 1024 * 1024 else 1
    nrows = n_dst // halves
    e_tile = n_edges // _NS
    steps = e_tile // 128
    rows_tile = nrows // _NS
    out_chunks = rows_tile // 128
    nacc = nrows + 8
    mesh = plsc.VectorSubcoreMesh(core_axis_name="c", subcore_axis_name="s")

    out_type = []
    for _ in range(nconv):
        out_type.append(jax.ShapeDtypeStruct((_NC, n_dst, _HALF), jnp.float32))
        out_type.append(jax.ShapeDtypeStruct((n_dst, _HALF), jnp.float32))

    scratch = [pltpu.VMEM_SHARED((nacc, _HALF), jnp.float32)]
    if halves == 1:
        scratch.append(pltpu.VMEM_SHARED((nacc, _HALF), jnp.float32))
    scratch += [
        pltpu.VMEM((e_tile,), jnp.int32),
        pltpu.VMEM((steps, 128), jnp.int32),
        pltpu.VMEM((128, _HALF), jnp.float32),
        pltpu.VMEM((128, _HALF), jnp.float32),
        pltpu.VMEM((128, _HALF), jnp.float32),
        pltpu.VMEM((128, _HALF), jnp.float32),
        pltpu.SemaphoreType.DMA,
        pltpu.SemaphoreType.DMA,
    ]

    @functools.partial(pl.kernel, out_type=tuple(out_type), mesh=mesh,
                       scratch_types=scratch)
    def kern(*args):
        ins = args[:3 * nconv]
        outs = args[3 * nconv:5 * nconv]
        rest = args[5 * nconv:]
        if halves == 1:
            acc, cacc, sidx, didx, rows, rowsb, stage, ones, sem, semb = rest
        else:
            acc, sidx, didx, rows, rowsb, stage, ones, sem, semb = rest
            cacc = acc
        c = lax.axis_index("c")
        s = lax.axis_index("s")
        _fill_zero_ones(stage, ones)
        base = s * rows_tile

        def zero_acc(buf):
            for k in range(out_chunks):
                pltpu.sync_copy(stage, buf.at[pl.ds(base + k * 128, 128)])

            @pl.when(s == 0)
            def _():
                pltpu.sync_copy(stage.at[pl.ds(0, 8)], buf.at[pl.ds(nrows, 8)])

        def load_sidx(srcs):
            pltpu.sync_copy(srcs.at[pl.ds(s * e_tile, e_tile)], sidx)

            def addc(i, _):
                sidx[pl.ds(i * 16, 16)] = sidx[pl.ds(i * 16, 16)] + c
                return 0
            lax.fori_loop(0, e_tile // 16, addc, 0)

        def load_didx(dst2, rbase, do_clamp):
            pltpu.sync_copy(dst2.at[pl.ds(s * steps, steps)], didx)
            if do_clamp:
                _clamp_dst(didx, steps, rbase, nrows)

        def pipe(xs):
            pltpu.async_copy(xs.at[sidx.at[pl.ds(0, 128)]], rows, sem)

            def step2(jj, _):
                jb = 2 * jj + 1
                pltpu.async_copy(
                    xs.at[sidx.at[pl.ds(jb * 128, 128)]], rowsb, semb)
                pltpu.make_async_copy(
                    xs.at[sidx.at[pl.ds(0, 128)]], rows, sem).wait()
                pltpu.sync_copy(rows, acc.at[didx.at[2 * jj]], add=True)

                @pl.when(jb + 1 < steps)
                def _():
                    pltpu.async_copy(
                        xs.at[sidx.at[pl.ds((jb + 1) * 128, 128)]], rows, sem)
                pltpu.make_async_copy(
                    xs.at[sidx.at[pl.ds(0, 128)]], rowsb, semb).wait()
                pltpu.sync_copy(rowsb, acc.at[didx.at[jb]], add=True)
                return 0
            lax.fori_loop(0, steps // 2, step2, 0)

        def count_scatter():
            def cstep(j, _):
                pltpu.sync_copy(ones, cacc.at[didx.at[j]], add=True)
                return 0
            lax.fori_loop(0, steps, cstep, 0)

        def copy_out(buf, dst_ref, rowbase):
            for k in range(out_chunks):
                off = base + k * 128
                pltpu.sync_copy(buf.at[pl.ds(off, 128)], rows)
                pltpu.sync_copy(rows, dst_ref.at[pl.ds(rowbase + off, 128)])

        for v in range(nconv):
            xs, srcs, dst2 = ins[3 * v:3 * v + 3]
            sums, cnts = outs[2 * v:2 * v + 2]
            if halves == 1:
                zero_acc(acc)
                zero_acc(cacc)
                load_sidx(srcs)
                load_didx(dst2, 0, False)
                plsc.subcore_barrier()
                count_scatter()
                pipe(xs)
                plsc.subcore_barrier()
                copy_out(acc, sums.at[c], 0)

                @pl.when(c == 0)
                def _():
                    copy_out(cacc, cnts, 0)
            else:
                for h in range(2):
                    zero_acc(acc)
                    load_sidx(srcs)
                    load_didx(dst2, h * nrows, True)
                    plsc.subcore_barrier()
                    pipe(xs)
                    plsc.subcore_barrier()
                    copy_out(acc, sums.at[c], h * nrows)
                zero_acc(acc)
                load_didx(dst2, c * nrows, True)
                plsc.subcore_barrier()
                count_scatter()
                plsc.subcore_barrier()
                copy_out(acc, cnts, c * nrows)

    return kern


def _seg_sum_group(convs, n_dst):
    n_edges = convs[0][1].shape[1]
    args = []
    for x_src, ei in convs:
        n_src = x_src.shape[0]
        args.append(x_src.reshape(_NC * n_src, _HALF))
        args.append(ei[0] * _NC)
        args.append(ei[1].reshape(n_edges // 128, 128))
    outs = _make_seg_fused(n_dst, n_edges, len(convs))(*args)
    return [(outs[2 * v], outs[2 * v + 1]) for v in range(len(convs))]


@functools.lru_cache(maxsize=None)
def _make_dense(lq, lk, bq, split):
    grid = (lq // bq,)
    f32 = jnp.float32
    w = _DIM // split

    bf16 = jnp.bfloat16

    def body(hq, hk, suma, cnta, sumb, cntb,
             aq_w, aq_b, ak_w, ak_b, av_w, av_b, wo_t, wo_b,
             wla_t, wlb_t, wr_t, b_sage,
             o_ref, kp, vp):
        i = pl.program_id(0)

        @pl.when(i == 0)
        def _():
            hkv = hk[...].astype(bf16)
            kp[...] = (jnp.dot(hkv, ak_w[...], preferred_element_type=f32)
                       + ak_b[...]).astype(bf16)
            vp[...] = (jnp.dot(hkv, av_w[...], preferred_element_type=f32)
                       + av_b[...]).astype(bf16)

        h = hq[...]
        hb = h.astype(bf16)
        qp = (jnp.dot(hb, aq_w[...], preferred_element_type=f32)
              + aq_b[...]).astype(bf16)
        scale = 1.0 / math.sqrt(_DH)
        heads = []
        for hd in range(_H):
            qh = qp[:, hd * _DH:(hd + 1) * _DH]
            kh = kp[:, hd * _DH:(hd + 1) * _DH]
            vh = vp[:, hd * _DH:(hd + 1) * _DH]
            sc = lax.dot_general(qh, kh, (((1,), (1,)), ((), ())),
                                 preferred_element_type=f32) * scale
            m = jnp.max(sc, axis=-1, keepdims=True)
            p = jnp.exp(sc - m)
            p = (p / jnp.sum(p, axis=-1, keepdims=True)).astype(bf16)
            heads.append(jnp.dot(p, vh, preferred_element_type=f32))
        att = jnp.concatenate(heads, axis=1).astype(bf16)
        att = jnp.dot(att, wo_t[...], preferred_element_type=f32) + wo_b[...]

        ca = 1.0 / jnp.maximum(cnta[:, 0:1], 1.0)
        cb = 1.0 / jnp.maximum(cntb[:, 0:1], 1.0)
        up = jnp.dot(hb, wr_t[...], preferred_element_type=f32) + b_sage[...]
        for q in range(split):
            up = up + jnp.dot((suma[q] * ca).astype(bf16),
                              wla_t[q * w:(q + 1) * w, :],
                              preferred_element_type=f32)
            up = up + jnp.dot((sumb[q] * cb).astype(bf16),
                              wlb_t[q * w:(q + 1) * w, :],
                              preferred_element_type=f32)

        o_ref[...] = 2.0 * h + att + up

    full = lambda shape: pl.BlockSpec(shape, lambda i: (0,) * len(shape))
    in_specs = [
        pl.BlockSpec((bq, _DIM), lambda i: (i, 0)),
        full((lk, _DIM)),
        pl.BlockSpec((split, bq, w), lambda i: (0, i, 0)),
        pl.BlockSpec((bq, _HALF), lambda i: (i, 0)),
        pl.BlockSpec((split, bq, w), lambda i: (0, i, 0)),
        pl.BlockSpec((bq, _HALF), lambda i: (i, 0)),
        full((_DIM, _DIM)), full((1, _DIM)),
        full((_DIM, _DIM)), full((1, _DIM)),
        full((_DIM, _DIM)), full((1, _DIM)),
        full((_DIM, _DIM)), full((1, _DIM)),
        full((_DIM, _DIM)), full((_DIM, _DIM)),
        full((_DIM, _DIM)), full((1, _DIM)),
    ]
    return pl.pallas_call(
        body,
        grid=grid,
        in_specs=in_specs,
        out_specs=pl.BlockSpec((bq, _DIM), lambda i: (i, 0)),
        out_shape=jax.ShapeDtypeStruct((lq, _DIM), f32),
        scratch_shapes=[
            pltpu.VMEM((lk, _DIM), jnp.bfloat16),
            pltpu.VMEM((lk, _DIM), jnp.bfloat16),
        ],
    )


def _fused_proj(base_w, base_b, et_w, et_b, se):
    a = jnp.dot(et_w.T, base_w.T)
    b = jnp.dot(se @ et_w.T + et_b, base_w.T) + base_b
    return a, b.reshape(1, _DIM)


def kernel(h_x5text, h_x20text, h_x5image, h_x20image,
           sage_Wl, sage_bl, sage_Wr,
           scale_emb, qW, qb, kW, kb,
           in_proj_w, in_proj_b, out_proj_w, out_proj_b,
           ei_self_x5text, ei_self_x20text, ei_self_x5image, ei_self_x20image,
           ei_x20i_img2txt, ei_x20i_txt2img, ei_x5i_img2txt, ei_x5i_txt2img):
    n5 = h_x5text.shape[0]
    n20 = h_x20text.shape[0]

    (s0, c0), (s2, c2), (s6, c6), (s7, c7) = _seg_sum_group(
        [(h_x5text, ei_self_x5text), (h_x5image, ei_self_x5image),
         (h_x5image, ei_x5i_img2txt), (h_x5text, ei_x5i_txt2img)], n5)
    (s1, c1), (s3, c3), (s4, c4), (s5, c5) = _seg_sum_group(
        [(h_x20text, ei_self_x20text), (h_x20image, ei_self_x20image),
         (h_x20image, ei_x20i_img2txt), (h_x20text, ei_x20i_txt2img)], n20)

    Wq, Wk, Wv = in_proj_w[:_DIM], in_proj_w[_DIM:2 * _DIM], in_proj_w[2 * _DIM:]
    bq_, bk_, bv_ = in_proj_b[:_DIM], in_proj_b[_DIM:2 * _DIM], in_proj_b[2 * _DIM:]
    wo_t = out_proj_w.T
    wo_b = out_proj_b.reshape(1, _DIM)

    def dense(hq, hk, et, qs, ks, sa, ca, sb, cb, wla, wlb, wra, wrb, bla, blb, bq_blk):
        aq_w, aq_b = _fused_proj(Wq, bq_, qW[et], qb[et], scale_emb[qs])
        ak_w, ak_b = _fused_proj(Wk, bk_, kW[et], kb[et], scale_emb[ks])
        av_w, av_b = _fused_proj(Wv, bv_, kW[et], kb[et], scale_emb[ks])
        call = _make_dense(hq.shape[0], hk.shape[0], bq_blk, sa.shape[0])
        bf = jnp.bfloat16
        return call(hq, hk, sa, ca, sb, cb,
                    aq_w.astype(bf), aq_b, ak_w.astype(bf), ak_b,
                    av_w.astype(bf), av_b, wo_t.astype(bf), wo_b,
                    wla.T.astype(bf), wlb.T.astype(bf),
                    (wra + wrb).T.astype(bf),
                    (bla + blb).reshape(1, _DIM))

    n_x5text = dense(h_x5text, h_x20text, 2, 0, 1, s0, c0, s6, c6,
                     sage_Wl[0], sage_Wl[6], sage_Wr[0], sage_Wr[6],
                     sage_bl[0], sage_bl[6], 512)
    n_x20text = dense(h_x20text, h_x5text, 3, 1, 0, s1, c1, s4, c4,
                      sage_Wl[1], sage_Wl[4], sage_Wr[1], sage_Wr[4],
                      sage_bl[1], sage_bl[4], 256)
    n_x5image = dense(h_x5image, h_x20image, 0, 0, 1, s2, c2, s7, c7,
                      sage_Wl[2], sage_Wl[7], sage_Wr[2], sage_Wr[7],
                      sage_bl[2], sage_bl[7], 512)
    n_x20image = dense(h_x20image, h_x5image, 1, 1, 0, s3, c3, s5, c5,
                       sage_Wl[3], sage_Wl[5], sage_Wr[3], sage_Wr[5],
                       sage_bl[3], sage_bl[5], 256)
    return (n_x5text, n_x20text, n_x5image, n_x20image)


# speedup vs baseline: 1.0699x; 1.0699x over previous
import functools
import math

import jax
import jax.numpy as jnp
from jax import lax
from jax.experimental import pallas as pl
from jax.experimental.pallas import tpu as pltpu
from jax.experimental.pallas import tpu_sc as plsc

_DIM = 256
_H = 2
_DH = _DIM // _H
_HALF = 128
_NC = 2
_NS = 16


def _fill_zero_ones(stage, ones):
    zero16 = jnp.zeros((16,), jnp.float32)
    one16 = jnp.ones((16,), jnp.float32)

    def fill(r, _):
        for g in range(_HALF // 16):
            stage[r, pl.ds(g * 16, 16)] = zero16
            if ones is not None:
                ones[r, pl.ds(g * 16, 16)] = one16
        return 0
    lax.fori_loop(0, 128, fill, 0)


def _clamp_dst(didx, steps, rbase, nrows):
    def clampd(j, _):
        for g in range(128 // 16):
            v = didx[j, pl.ds(g * 16, 16)] - rbase
            ok = (v >= 0) & (v < nrows)
            didx[j, pl.ds(g * 16, 16)] = jnp.where(ok, v, nrows)
        return 0
    lax.fori_loop(0, steps, clampd, 0)


@functools.lru_cache(maxsize=None)
def _make_seg_sum(nrows, rbase, clamp, n_edges, with_counts):
    e_tile = n_edges // _NS
    steps = e_tile // 128
    rows_tile = nrows // _NS
    out_chunks = rows_tile // 128
    nacc = nrows + 8 if clamp else nrows
    mesh = plsc.VectorSubcoreMesh(core_axis_name="c", subcore_axis_name="s")

    out_type = [jax.ShapeDtypeStruct((_NC, nrows, _HALF), jnp.float32)]
    scratch = [
        pltpu.VMEM_SHARED((nacc, _HALF), jnp.float32),
        pltpu.VMEM((e_tile,), jnp.int32),
        pltpu.VMEM((steps, 128), jnp.int32),
        pltpu.VMEM((128, _HALF), jnp.float32),
        pltpu.VMEM((128, _HALF), jnp.float32),
        pltpu.VMEM((128, _HALF), jnp.float32),
        pltpu.VMEM((128, _HALF), jnp.float32),
        pltpu.SemaphoreType.DMA,
        pltpu.SemaphoreType.DMA,
        pltpu.SemaphoreType.DMA,
        pltpu.SemaphoreType.DMA,
    ]
    if with_counts:
        out_type.append(jax.ShapeDtypeStruct((nrows, _HALF), jnp.float32))
        scratch.insert(1, pltpu.VMEM_SHARED((nacc, _HALF), jnp.float32))

    @functools.partial(pl.kernel, out_type=tuple(out_type), mesh=mesh,
                       scratch_types=scratch)
    def kern(xs, srcs, dst2, sums, *rest):
        if with_counts:
            cnts, acc, cacc, sidx, didx, r0, r1, r2, stage, m0, m1, m2, m3 = rest
        else:
            (acc, sidx, didx, r0, r1, r2, stage, m0, m1, m2, m3) = rest
            cnts = cacc = None
        bufs = (r0, r1, r2, stage)
        sems = (m0, m1, m2, m3)
        rows = r0
        c = lax.axis_index("c")
        s = lax.axis_index("s")
        _fill_zero_ones(stage, None)

        base = s * rows_tile
        for k in range(out_chunks):
            pltpu.sync_copy(stage, acc.at[pl.ds(base + k * 128, 128)])
            if with_counts:
                pltpu.sync_copy(stage, cacc.at[pl.ds(base + k * 128, 128)])

        if clamp:
            @pl.when(s == 0)
            def _():
                pltpu.sync_copy(stage.at[pl.ds(0, 8)], acc.at[pl.ds(nrows, 8)])

        pltpu.sync_copy(srcs.at[pl.ds(s * e_tile, e_tile)], sidx)
        pltpu.sync_copy(dst2.at[pl.ds(s * steps, steps)], didx)

        def addc(i, _):
            sidx[pl.ds(i * 16, 16)] = sidx[pl.ds(i * 16, 16)] + c
            return 0
        lax.fori_loop(0, e_tile // 16, addc, 0)

        if clamp:
            _clamp_dst(didx, steps, rbase, nrows)

        if with_counts:
            _fill_zero_ones(rows, rows)
            plsc.subcore_barrier()

            def cstep(j, _):
                pltpu.sync_copy(rows, cacc.at[didx.at[j]], add=True)
                return 0
            lax.fori_loop(0, steps, cstep, 0)

        plsc.subcore_barrier()

        for t in range(3):
            pltpu.async_copy(xs.at[sidx.at[pl.ds(t * 128, 128)]],
                             bufs[t], sems[t])

        def step4(q, _):
            for t in range(4):
                j = 4 * q + t
                pltpu.make_async_copy(
                    xs.at[sidx.at[pl.ds(0, 128)]], bufs[t], sems[t]).wait()
                pltpu.sync_copy(bufs[t], acc.at[didx.at[j]], add=True)
                nj = j + 3

                @pl.when(nj < steps)
                def _():
                    pltpu.async_copy(
                        xs.at[sidx.at[pl.ds(nj * 128, 128)]],
                        bufs[(t + 3) % 4], sems[(t + 3) % 4])
            return 0
        lax.fori_loop(0, steps // 4, step4, 0)

        plsc.subcore_barrier()

        for k in range(out_chunks):
            off = base + k * 128
            pltpu.sync_copy(acc.at[pl.ds(off, 128)], rows)
            pltpu.sync_copy(rows, sums.at[c, pl.ds(off, 128)])

        if with_counts:
            @pl.when(c == 0)
            def _():
                for k in range(out_chunks):
                    off = base + k * 128
                    pltpu.sync_copy(cacc.at[pl.ds(off, 128)], rows)
                    pltpu.sync_copy(rows, cnts.at[pl.ds(off, 128)])

    return kern


@functools.lru_cache(maxsize=None)
def _make_counts(n_dst, n_edges):
    e_tile = n_edges // _NS
    steps = e_tile // 128
    nrows = n_dst // _NC
    rows_tile = nrows // _NS
    out_chunks = rows_tile // 128
    nacc = nrows + 8
    mesh = plsc.VectorSubcoreMesh(core_axis_name="c", subcore_axis_name="s")

    @functools.partial(
        pl.kernel,
        out_type=jax.ShapeDtypeStruct((n_dst, _HALF), jnp.float32),
        mesh=mesh,
        scratch_types=[
            pltpu.VMEM_SHARED((nacc, _HALF), jnp.float32),
            pltpu.VMEM((steps, 128), jnp.int32),
            pltpu.VMEM((128, _HALF), jnp.float32),
            pltpu.VMEM((128, _HALF), jnp.float32),
        ],
    )
    def kern(dst2, cnts, cacc, didx, stage, ones):
        c = lax.axis_index("c")
        s = lax.axis_index("s")
        _fill_zero_ones(stage, ones)

        base = s * rows_tile
        for k in range(out_chunks):
            pltpu.sync_copy(stage, cacc.at[pl.ds(base + k * 128, 128)])

        @pl.when(s == 0)
        def _():
            pltpu.sync_copy(stage.at[pl.ds(0, 8)], cacc.at[pl.ds(nrows, 8)])

        pltpu.sync_copy(dst2.at[pl.ds(s * steps, steps)], didx)
        _clamp_dst(didx, steps, c * nrows, nrows)

        plsc.subcore_barrier()

        def step(j, _):
            pltpu.sync_copy(ones, cacc.at[didx.at[j]], add=True)
            return 0
        lax.fori_loop(0, steps, step, 0)

        plsc.subcore_barrier()

        for k in range(out_chunks):
            off = base + k * 128
            pltpu.sync_copy(cacc.at[pl.ds(off, 128)], ones)
            pltpu.sync_copy(ones, cnts.at[pl.ds(c * nrows + off, 128)])

    return kern


def _seg_sum(x_src, ei, n_dst):
    n_src = x_src.shape[0]
    n_edges = ei.shape[1]
    xs = x_src.reshape(_NC * n_src, _HALF)
    srcs = ei[0] * _NC
    dst2 = ei[1].reshape(n_edges // 128, 128)
    halves = 2 if n_dst * _HALF * 4 > 3 * 1024 * 1024 else 1
    if halves == 1:
        return _make_seg_sum(n_dst, 0, False, n_edges, True)(xs, srcs, dst2)
    nrows = n_dst // halves
    parts = [_make_seg_sum(nrows, h * nrows, True, n_edges, False)(xs, srcs, dst2)
             for h in range(halves)]
    sums = jnp.concatenate([p[0] for p in parts], axis=1)
    cnts = _make_counts(n_dst, n_edges)(dst2)
    return sums, cnts


@functools.lru_cache(maxsize=None)
def _make_dense(lq, lk, bq, split):
    grid = (lq // bq,)
    f32 = jnp.float32
    w = _DIM // split

    bf16 = jnp.bfloat16

    def body(hq, hk, suma, cnta, sumb, cntb,
             aq_w, aq_b, ak_w, ak_b, av_w, av_b, wo_t, wo_b,
             wla_t, wlb_t, wr_t, b_sage,
             o_ref, kp, vp):
        i = pl.program_id(0)

        @pl.when(i == 0)
        def _():
            hkv = hk[...].astype(bf16)
            kp[...] = (jnp.dot(hkv, ak_w[...], preferred_element_type=f32)
                       + ak_b[...]).astype(bf16)
            vp[...] = (jnp.dot(hkv, av_w[...], preferred_element_type=f32)
                       + av_b[...]).astype(bf16)

        h = hq[...]
        hb = h.astype(bf16)
        qp = (jnp.dot(hb, aq_w[...], preferred_element_type=f32)
              + aq_b[...]).astype(bf16)
        scale = 1.0 / math.sqrt(_DH)
        heads = []
        for hd in range(_H):
            qh = qp[:, hd * _DH:(hd + 1) * _DH]
            kh = kp[:, hd * _DH:(hd + 1) * _DH]
            vh = vp[:, hd * _DH:(hd + 1) * _DH]
            sc = lax.dot_general(qh, kh, (((1,), (1,)), ((), ())),
                                 preferred_element_type=f32) * scale
            m = jnp.max(sc, axis=-1, keepdims=True)
            p = jnp.exp(sc - m)
            p = (p / jnp.sum(p, axis=-1, keepdims=True)).astype(bf16)
            heads.append(jnp.dot(p, vh, preferred_element_type=f32))
        att = jnp.concatenate(heads, axis=1).astype(bf16)
        att = jnp.dot(att, wo_t[...], preferred_element_type=f32) + wo_b[...]

        ca = 1.0 / jnp.maximum(cnta[:, 0:1], 1.0)
        cb = 1.0 / jnp.maximum(cntb[:, 0:1], 1.0)
        up = jnp.dot(hb, wr_t[...], preferred_element_type=f32) + b_sage[...]
        for q in range(split):
            up = up + jnp.dot((suma[q] * ca).astype(bf16),
                              wla_t[q * w:(q + 1) * w, :],
                              preferred_element_type=f32)
            up = up + jnp.dot((sumb[q] * cb).astype(bf16),
                              wlb_t[q * w:(q + 1) * w, :],
                              preferred_element_type=f32)

        o_ref[...] = 2.0 * h + att + up

    full = lambda shape: pl.BlockSpec(shape, lambda i: (0,) * len(shape))
    in_specs = [
        pl.BlockSpec((bq, _DIM), lambda i: (i, 0)),
        full((lk, _DIM)),
        pl.BlockSpec((split, bq, w), lambda i: (0, i, 0)),
        pl.BlockSpec((bq, _HALF), lambda i: (i, 0)),
        pl.BlockSpec((split, bq, w), lambda i: (0, i, 0)),
        pl.BlockSpec((bq, _HALF), lambda i: (i, 0)),
        full((_DIM, _DIM)), full((1, _DIM)),
        full((_DIM, _DIM)), full((1, _DIM)),
        full((_DIM, _DIM)), full((1, _DIM)),
        full((_DIM, _DIM)), full((1, _DIM)),
        full((_DIM, _DIM)), full((_DIM, _DIM)),
        full((_DIM, _DIM)), full((1, _DIM)),
    ]
    return pl.pallas_call(
        body,
        grid=grid,
        in_specs=in_specs,
        out_specs=pl.BlockSpec((bq, _DIM), lambda i: (i, 0)),
        out_shape=jax.ShapeDtypeStruct((lq, _DIM), f32),
        scratch_shapes=[
            pltpu.VMEM((lk, _DIM), jnp.bfloat16),
            pltpu.VMEM((lk, _DIM), jnp.bfloat16),
        ],
    )


def _fused_proj(base_w, base_b, et_w, et_b, se):
    a = jnp.dot(et_w.T, base_w.T)
    b = jnp.dot(se @ et_w.T + et_b, base_w.T) + base_b
    return a, b.reshape(1, _DIM)


def kernel(h_x5text, h_x20text, h_x5image, h_x20image,
           sage_Wl, sage_bl, sage_Wr,
           scale_emb, qW, qb, kW, kb,
           in_proj_w, in_proj_b, out_proj_w, out_proj_b,
           ei_self_x5text, ei_self_x20text, ei_self_x5image, ei_self_x20image,
           ei_x20i_img2txt, ei_x20i_txt2img, ei_x5i_img2txt, ei_x5i_txt2img):
    n5 = h_x5text.shape[0]
    n20 = h_x20text.shape[0]

    s0, c0 = _seg_sum(h_x5text, ei_self_x5text, n5)
    s1, c1 = _seg_sum(h_x20text, ei_self_x20text, n20)
    s2, c2 = _seg_sum(h_x5image, ei_self_x5image, n5)
    s3, c3 = _seg_sum(h_x20image, ei_self_x20image, n20)
    s4, c4 = _seg_sum(h_x20image, ei_x20i_img2txt, n20)
    s5, c5 = _seg_sum(h_x20text, ei_x20i_txt2img, n20)
    s6, c6 = _seg_sum(h_x5image, ei_x5i_img2txt, n5)
    s7, c7 = _seg_sum(h_x5text, ei_x5i_txt2img, n5)

    Wq, Wk, Wv = in_proj_w[:_DIM], in_proj_w[_DIM:2 * _DIM], in_proj_w[2 * _DIM:]
    bq_, bk_, bv_ = in_proj_b[:_DIM], in_proj_b[_DIM:2 * _DIM], in_proj_b[2 * _DIM:]
    wo_t = out_proj_w.T
    wo_b = out_proj_b.reshape(1, _DIM)

    def dense(hq, hk, et, qs, ks, sa, ca, sb, cb, wla, wlb, wra, wrb, bla, blb, bq_blk):
        aq_w, aq_b = _fused_proj(Wq, bq_, qW[et], qb[et], scale_emb[qs])
        ak_w, ak_b = _fused_proj(Wk, bk_, kW[et], kb[et], scale_emb[ks])
        av_w, av_b = _fused_proj(Wv, bv_, kW[et], kb[et], scale_emb[ks])
        call = _make_dense(hq.shape[0], hk.shape[0], bq_blk, sa.shape[0])
        bf = jnp.bfloat16
        return call(hq, hk, sa, ca, sb, cb,
                    aq_w.astype(bf), aq_b, ak_w.astype(bf), ak_b,
                    av_w.astype(bf), av_b, wo_t.astype(bf), wo_b,
                    wla.T.astype(bf), wlb.T.astype(bf),
                    (wra + wrb).T.astype(bf),
                    (bla + blb).reshape(1, _DIM))

    n_x5text = dense(h_x5text, h_x20text, 2, 0, 1, s0, c0, s6, c6,
                     sage_Wl[0], sage_Wl[6], sage_Wr[0], sage_Wr[6],
                     sage_bl[0], sage_bl[6], 512)
    n_x20text = dense(h_x20text, h_x5text, 3, 1, 0, s1, c1, s4, c4,
                      sage_Wl[1], sage_Wl[4], sage_Wr[1], sage_Wr[4],
                      sage_bl[1], sage_bl[4], 256)
    n_x5image = dense(h_x5image, h_x20image, 0, 0, 1, s2, c2, s7, c7,
                      sage_Wl[2], sage_Wl[7], sage_Wr[2], sage_Wr[7],
                      sage_bl[2], sage_bl[7], 512)
    n_x20image = dense(h_x20image, h_x5image, 1, 1, 0, s3, c3, s5, c5,
                       sage_Wl[3], sage_Wl[5], sage_Wr[3], sage_Wr[5],
                       sage_bl[3], sage_bl[5], 256)
    return (n_x5text, n_x20text, n_x5image, n_x20image)
